# baseline (speedup 1.0000x reference)
import jax
import jax.numpy as jnp
from jax.experimental import pallas as pl

HIDDEN = 2048
N_EXPERTS = 64
BLOCK_M = 512


def _router_kernel(hs_ref, w_ref, out_ref):
    out_ref[...] = jax.lax.dot_general(
        hs_ref[...],
        w_ref[...],
        dimension_numbers=(((1,), (1,)), ((), ())),
        preferred_element_type=jnp.float32,
    )


def kernel(hidden_states, weight):
    hs = hidden_states.reshape(-1, HIDDEN).astype(jnp.float32)
    w = weight.astype(jnp.float32)
    m = hs.shape[0]
    return pl.pallas_call(
        _router_kernel,
        grid=(m // BLOCK_M,),
        in_specs=[
            pl.BlockSpec((BLOCK_M, HIDDEN), lambda i: (i, 0)),
            pl.BlockSpec((N_EXPERTS, HIDDEN), lambda i: (0, 0)),
        ],
        out_specs=pl.BlockSpec((BLOCK_M, N_EXPERTS), lambda i: (i, 0)),
        out_shape=jax.ShapeDtypeStruct((m, N_EXPERTS), jnp.float32),
    )(hs, w)


# speedup vs baseline: 1.0924x; 1.0924x over previous
import jax
import jax.numpy as jnp
from jax.experimental import pallas as pl

HIDDEN = 2048
N_EXPERTS = 64
BLOCK_M = 2048


def _router_kernel(hs_ref, w_ref, out_ref):
    out_ref[...] = jax.lax.dot_general(
        hs_ref[...],
        w_ref[...],
        dimension_numbers=(((1,), (1,)), ((), ())),
        preferred_element_type=jnp.float32,
    )


def kernel(hidden_states, weight):
    hs = hidden_states.reshape(-1, HIDDEN).astype(jnp.float32)
    w = weight.astype(jnp.float32)
    m = hs.shape[0]
    return pl.pallas_call(
        _router_kernel,
        grid=(m // BLOCK_M,),
        in_specs=[
            pl.BlockSpec((BLOCK_M, HIDDEN), lambda i: (i, 0)),
            pl.BlockSpec((N_EXPERTS, HIDDEN), lambda i: (0, 0)),
        ],
        out_specs=pl.BlockSpec((BLOCK_M, N_EXPERTS), lambda i: (i, 0)),
        out_shape=jax.ShapeDtypeStruct((m, N_EXPERTS), jnp.float32),
    )(hs, w)


# speedup vs baseline: 1.1241x; 1.0289x over previous
import functools

import jax
import jax.numpy as jnp
from jax.experimental import pallas as pl

HIDDEN = 2048
N_EXPERTS = 64
NSPLIT = 4
BLOCK_M = 256


def _router_kernel(*refs):
    w_ref = refs[NSPLIT]
    out_ref = refs[NSPLIT + 1]
    w = w_ref[...]
    for j in range(NSPLIT):
        out_ref[j * BLOCK_M:(j + 1) * BLOCK_M, :] = jax.lax.dot_general(
            refs[j][...],
            w,
            dimension_numbers=(((1,), (1,)), ((), ())),
            preferred_element_type=jnp.float32,
        )


def _hs_index_map(i, j):
    return (NSPLIT * i + j, 0)


def kernel(hidden_states, weight):
    hs = hidden_states.reshape(-1, HIDDEN).astype(jnp.float32)
    w = weight.astype(jnp.float32)
    m = hs.shape[0]
    rows_per_step = NSPLIT * BLOCK_M
    grid = (m // rows_per_step,)
    in_specs = [
        pl.BlockSpec((BLOCK_M, HIDDEN), functools.partial(_hs_index_map, j=j))
        for j in range(NSPLIT)
    ]
    in_specs.append(pl.BlockSpec((N_EXPERTS, HIDDEN), lambda i: (0, 0)))
    return pl.pallas_call(
        _router_kernel,
        grid=grid,
        in_specs=in_specs,
        out_specs=pl.BlockSpec((rows_per_step, N_EXPERTS), lambda i: (i, 0)),
        out_shape=jax.ShapeDtypeStruct((m, N_EXPERTS), jnp.float32),
    )(*([hs] * NSPLIT), w)
